# baseline (speedup 1.0000x reference)
import functools

import jax
import jax.numpy as jnp
from jax import lax
from jax.experimental import pallas as pl
from jax.experimental.pallas import tpu as pltpu
from jax.experimental.pallas import tpu_sc as plsc

N_ATOMS = 50000
D_OUT = 384
CH = 80
NCH = 20
APW = CH * NCH


def _make_kernel(nc: int, ns: int):
    mesh = plsc.VectorSubcoreMesh(core_axis_name="c", subcore_axis_name="s")

    @functools.partial(
        pl.kernel,
        mesh=mesh,
        compiler_params=pltpu.CompilerParams(needs_layout_passes=False),
        out_type=jax.ShapeDtypeStruct((N_ATOMS, D_OUT), jnp.float32),
        scratch_types=[
            pltpu.VMEM((APW,), jnp.int32),
            pltpu.VMEM((APW,), jnp.int32),
            pltpu.VMEM((APW,), jnp.int32),
            pltpu.VMEM((APW,), jnp.int32),
            pltpu.VMEM((128 * 128,), jnp.float32),
            pltpu.VMEM((64 * 128,), jnp.float32),
            pltpu.VMEM((32 * 64,), jnp.float32),
            pltpu.VMEM((CH, D_OUT), jnp.float32),
            pltpu.VMEM((CH, D_OUT), jnp.float32),
            pltpu.VMEM((CH, 128), jnp.float32),
            pltpu.VMEM((CH, 128), jnp.float32),
            pltpu.SemaphoreType.DMA,
            pltpu.SemaphoreType.DMA,
            pltpu.SemaphoreType.DMA,
            pltpu.SemaphoreType.DMA,
            pltpu.SemaphoreType.DMA,
        ],
    )
    def k(i1_hbm, i2_hbm, i3_hbm, i4_hbm, t1_hbm, t2_hbm, t3_hbm, t4_hbm,
          out_hbm, i1_v, i2_v, i3_v, i4_v, t1_v, t2_v, t3_v,
          comb_a, comb_b, buf4_a, buf4_b,
          isem, gsem_a, gsem_b, wsem_a, wsem_b):
        wid = lax.axis_index("s") * nc + lax.axis_index("c")
        base = jnp.minimum(wid * APW, N_ATOMS - APW)

        comb = (comb_a, comb_b)
        buf4 = (buf4_a, buf4_b)
        gsem = (gsem_a, gsem_b)
        wsem = (wsem_a, wsem_b)

        cps = [
            pltpu.async_copy(i1_hbm.at[pl.ds(base, APW)], i1_v, isem),
            pltpu.async_copy(i2_hbm.at[pl.ds(base, APW)], i2_v, isem),
            pltpu.async_copy(i3_hbm.at[pl.ds(base, APW)], i3_v, isem),
            pltpu.async_copy(i4_hbm.at[pl.ds(base, APW)], i4_v, isem),
            pltpu.async_copy(t1_hbm, t1_v, isem),
            pltpu.async_copy(t2_hbm, t2_v, isem),
            pltpu.async_copy(t3_hbm, t3_v, isem),
        ]
        for cp in cps:
            cp.wait()

        def fire_t4(kk, b):
            off = kk * CH
            return pltpu.async_copy(t4_hbm.at[i4_v.at[pl.ds(off, CH)]],
                                    buf4[b], gsem[b])

        def fill(b, kk):
            off = kk * CH
            iota = lax.iota(jnp.int32, 16)

            def group(g, cc):
                gbase = off + 16 * g
                base1 = i1_v[pl.ds(gbase, 16)] * 128
                base2 = i2_v[pl.ds(gbase, 16)] * 128
                base3 = i3_v[pl.ds(gbase, 16)] * 64
                for l in range(16):
                    r = 16 * g + l
                    lane = jnp.full((16,), l, jnp.int32)
                    s1 = jnp.take(base1, lane) + iota
                    s2 = jnp.take(base2, lane) + iota
                    s3 = jnp.take(base3, lane) + iota
                    for s in range(8):
                        comb[b][r, pl.ds(16 * s, 16)] = \
                            plsc.load_gather(t1_v, [s1 + 16 * s])
                    for s in range(8):
                        comb[b][r, pl.ds(128 + 16 * s, 16)] = \
                            plsc.load_gather(t2_v, [s2 + 16 * s])
                    for s in range(4):
                        comb[b][r, pl.ds(256 + 16 * s, 16)] = \
                            plsc.load_gather(t3_v, [s3 + 16 * s])
                return cc

            lax.fori_loop(0, CH // 16, group, 0)

        def merge(b):
            def row(r, cc):
                for s in range(4):
                    comb[b][r, pl.ds(320 + 16 * s, 16)] = \
                        buf4[b][r, pl.ds(16 * s, 16)]
                return cc
            lax.fori_loop(0, CH, row, 0)

        def wait_gather(b):
            pltpu.make_async_copy(t4_hbm.at[i4_v.at[pl.ds(0, CH)]],
                                  buf4[b], gsem[b]).wait()

        def wait_write(b):
            pltpu.make_async_copy(comb[b], out_hbm.at[pl.ds(base, CH)],
                                  wsem[b]).wait()

        def fire_write(kk, b):
            pltpu.async_copy(comb[b], out_hbm.at[pl.ds(base + kk * CH, CH)],
                             wsem[b])

        fire_t4(0, 0)
        fire_t4(1, 1)

        def pair_body(kk2, carry):
            for b in (0, 1):
                c = 2 * kk2 + b

                @pl.when(kk2 > 0)
                def _():
                    wait_write(b)

                fill(b, c)
                wait_gather(b)
                merge(b)

                @pl.when(c + 2 < NCH)
                def _():
                    fire_t4(c + 2, b)

                fire_write(c, b)
            return carry

        lax.fori_loop(0, NCH // 2, pair_body, 0)
        wait_write(0)
        wait_write(1)

    return k


def kernel(atom_type_index, atom_code_index, residue_code_index,
           residue_sequence_index, atom_type_table, atom_code_table,
           residue_code_table, residue_index_table):
    i1 = atom_type_index.astype(jnp.int32)
    i2 = atom_code_index.astype(jnp.int32)
    i3 = residue_code_index.astype(jnp.int32)
    i4 = residue_sequence_index.astype(jnp.int32)
    t4p = jnp.pad(residue_index_table, ((0, 0), (0, 64)))
    info = plsc.get_sparse_core_info()
    k = _make_kernel(info.num_cores, info.num_subcores)
    return k(i1, i2, i3, i4, atom_type_table.reshape(-1),
             atom_code_table.reshape(-1), residue_code_table.reshape(-1),
             t4p)


# speedup vs baseline: 1.1188x; 1.1188x over previous
import functools

import jax
import jax.numpy as jnp
from jax import lax
from jax.experimental import pallas as pl
from jax.experimental.pallas import tpu as pltpu
from jax.experimental.pallas import tpu_sc as plsc

N_ATOMS = 50000
D_OUT = 384
CH = 80
NCH = 20
APW = CH * NCH


def _make_kernel(nc: int, ns: int):
    mesh = plsc.VectorSubcoreMesh(core_axis_name="c", subcore_axis_name="s")

    @functools.partial(
        pl.kernel,
        mesh=mesh,
        out_type=jax.ShapeDtypeStruct((N_ATOMS, D_OUT), jnp.float32),
        scratch_types=[
            pltpu.VMEM((APW,), jnp.int32),
            pltpu.VMEM((APW,), jnp.int32),
            pltpu.VMEM((APW,), jnp.int32),
            pltpu.VMEM((APW,), jnp.int32),
            pltpu.VMEM((128, 128), jnp.float32),
            pltpu.VMEM((64, 128), jnp.float32),
            pltpu.VMEM((32, 64), jnp.float32),
            pltpu.VMEM((CH, D_OUT), jnp.float32),
            pltpu.VMEM((CH, D_OUT), jnp.float32),
            pltpu.SemaphoreType.DMA,
            pltpu.SemaphoreType.DMA,
            pltpu.SemaphoreType.DMA,
            pltpu.SemaphoreType.DMA,
            pltpu.SemaphoreType.DMA,
        ],
    )
    def k(i1_hbm, i2_hbm, i3_hbm, i4_hbm, t1_hbm, t2_hbm, t3_hbm, t4_hbm,
          out_hbm, i1_v, i2_v, i3_v, i4_v, t1_v, t2_v, t3_v,
          comb_a, comb_b, isem, gsem_a, gsem_b, wsem_a, wsem_b):
        wid = lax.axis_index("s") * nc + lax.axis_index("c")
        base = jnp.minimum(wid * APW, N_ATOMS - APW)

        comb = (comb_a, comb_b)
        gsem = (gsem_a, gsem_b)
        wsem = (wsem_a, wsem_b)

        cps = [
            pltpu.async_copy(i1_hbm.at[pl.ds(base, APW)], i1_v, isem),
            pltpu.async_copy(i2_hbm.at[pl.ds(base, APW)], i2_v, isem),
            pltpu.async_copy(i3_hbm.at[pl.ds(base, APW)], i3_v, isem),
            pltpu.async_copy(i4_hbm.at[pl.ds(base, APW)], i4_v, isem),
            pltpu.async_copy(t1_hbm, t1_v, isem),
            pltpu.async_copy(t2_hbm, t2_v, isem),
            pltpu.async_copy(t3_hbm, t3_v, isem),
        ]
        for cp in cps:
            cp.wait()

        def fire_t4(kk, b):
            off = kk * CH
            return pltpu.async_copy(t4_hbm.at[i4_v.at[pl.ds(off, CH)]],
                                    comb[b].at[:, pl.ds(256, 128)], gsem[b])

        def fill_main(b, kk):
            off = kk * CH

            def group(g, cc):
                gbase = off + 16 * g
                iv1 = i1_v[pl.ds(gbase, 16)]
                iv2 = i2_v[pl.ds(gbase, 16)]
                for l in range(16):
                    r = 16 * g + l
                    i1r = iv1[l]
                    i2r = iv2[l]
                    for s in range(8):
                        comb[b][r, pl.ds(16 * s, 16)] = t1_v[i1r, pl.ds(16 * s, 16)]
                    for s in range(8):
                        comb[b][r, pl.ds(128 + 16 * s, 16)] = t2_v[i2r, pl.ds(16 * s, 16)]
                return cc

            lax.fori_loop(0, CH // 16, group, 0)

        def fill_t3(b, kk):
            off = kk * CH

            def group(g, cc):
                iv3 = i3_v[pl.ds(off + 16 * g, 16)]
                for l in range(16):
                    r = 16 * g + l
                    i3r = iv3[l]
                    for s in range(4):
                        comb[b][r, pl.ds(256 + 16 * s, 16)] = t3_v[i3r, pl.ds(16 * s, 16)]
                return cc

            lax.fori_loop(0, CH // 16, group, 0)

        def wait_write(b):
            pltpu.make_async_copy(comb[b], out_hbm.at[pl.ds(base, CH)],
                                  wsem[b]).wait()

        def chunk_step(b, c):
            @pl.when(c >= 2)
            def _():
                wait_write(b)

            g = fire_t4(c, b)
            fill_main(b, c)
            g.wait()
            fill_t3(b, c)
            pltpu.async_copy(comb[b], out_hbm.at[pl.ds(base + c * CH, CH)],
                             wsem[b])

        def pair_body(kk2, carry):
            chunk_step(0, 2 * kk2)
            chunk_step(1, 2 * kk2 + 1)
            return carry

        lax.fori_loop(0, NCH // 2, pair_body, 0)
        wait_write(0)
        wait_write(1)

    return k


def kernel(atom_type_index, atom_code_index, residue_code_index,
           residue_sequence_index, atom_type_table, atom_code_table,
           residue_code_table, residue_index_table):
    i1 = atom_type_index.astype(jnp.int32)
    i2 = atom_code_index.astype(jnp.int32)
    i3 = residue_code_index.astype(jnp.int32)
    i4 = residue_sequence_index.astype(jnp.int32)
    t4p = jnp.pad(residue_index_table, ((0, 0), (64, 0)))
    info = plsc.get_sparse_core_info()
    k = _make_kernel(info.num_cores, info.num_subcores)
    return k(i1, i2, i3, i4, atom_type_table, atom_code_table,
             residue_code_table, t4p)
